# Initial kernel scaffold; baseline (speedup 1.0000x reference)
import jax
import jax.numpy as jnp
from jax import lax
from jax.experimental import pallas as pl
from jax.experimental.pallas import tpu as pltpu
from jax.experimental.pallas import tpu_sc as plsc

VOCAB_SIZE = 100000
HIDDEN = 64
PAD = 0
N_TOKENS = 4096 * 200
LANES = 16
NUM_CORES = 2
NUM_SUBCORES = 16
NUM_WORKERS = NUM_CORES * NUM_SUBCORES

IDS_PER_GATHER = 128
GATHERS_PER_CHUNK = 8
CHUNK = IDS_PER_GATHER * GATHERS_PER_CHUNK
ROWS128 = N_TOKENS // IDS_PER_GATHER
ROWS128_PER_WORKER = ROWS128 // NUM_WORKERS
CHUNKS_PER_WORKER = ROWS128_PER_WORKER // GATHERS_PER_CHUNK
GROUPS_PER_CHUNK = CHUNK // LANES


def _sc_body(ids_hbm, table_hbm, out_hbm, mask_hbm, idx_v, rows_v, mask_v, sem):
    wid = lax.axis_index("c") * NUM_SUBCORES + lax.axis_index("s")

    def chunk_body(k, carry):
        row0 = wid * ROWS128_PER_WORKER + k * GATHERS_PER_CHUNK
        pltpu.sync_copy(ids_hbm.at[pl.ds(row0, GATHERS_PER_CHUNK)], idx_v)

        descs = []
        for j in range(GATHERS_PER_CHUNK):
            descs.append(
                pltpu.async_copy(
                    table_hbm.at[idx_v.at[j]],
                    rows_v.at[pl.ds(j * IDS_PER_GATHER, IDS_PER_GATHER)],
                    sem,
                )
            )
        for d in descs:
            d.wait()

        def group_body(g, carry2):
            j = g // (IDS_PER_GATHER // LANES)
            col = (g % (IDS_PER_GATHER // LANES)) * LANES
            idv = idx_v[j, pl.ds(col, LANES)]
            mvec = (idv != PAD).astype(jnp.int32)
            mask_v[j, pl.ds(col, LANES)] = mvec

            @pl.when(jnp.min(idv) == PAD)
            def _fixup():
                base = g * LANES
                for r in range(LANES):
                    @pl.when(idx_v[j, col + r] == PAD)
                    def _zero_row():
                        for c in range(HIDDEN // LANES):
                            rows_v[base + r, pl.ds(c * LANES, LANES)] = (
                                jnp.zeros((LANES,), jnp.float32)
                            )
            return carry2

        lax.fori_loop(0, GROUPS_PER_CHUNK, group_body, 0)

        pltpu.sync_copy(rows_v, out_hbm.at[pl.ds(row0 * IDS_PER_GATHER, CHUNK)])
        pltpu.sync_copy(mask_v, mask_hbm.at[pl.ds(row0, GATHERS_PER_CHUNK)])
        return carry

    lax.fori_loop(0, CHUNKS_PER_WORKER, chunk_body, 0)


@jax.jit
def kernel(input_ids, embed_weight):
    ids2d = input_ids.reshape(ROWS128, IDS_PER_GATHER)
    mesh = plsc.VectorSubcoreMesh(
        core_axis_name="c", subcore_axis_name="s",
        num_cores=NUM_CORES, num_subcores=NUM_SUBCORES,
    )
    out, mask2d = pl.kernel(
        _sc_body,
        out_type=[
            jax.ShapeDtypeStruct((N_TOKENS, HIDDEN), jnp.float32),
            jax.ShapeDtypeStruct((ROWS128, IDS_PER_GATHER), jnp.int32),
        ],
        mesh=mesh,
        scratch_types=[
            pltpu.VMEM((GATHERS_PER_CHUNK, IDS_PER_GATHER), jnp.int32),
            pltpu.VMEM((CHUNK, HIDDEN), jnp.float32),
            pltpu.VMEM((GATHERS_PER_CHUNK, IDS_PER_GATHER), jnp.int32),
            pltpu.SemaphoreType.DMA,
        ],
    )(ids2d, embed_weight)
    return (
        out.reshape(input_ids.shape[0], input_ids.shape[1], HIDDEN),
        mask2d.reshape(input_ids.shape),
    )


# speedup vs baseline: 3.9629x; 3.9629x over previous
import jax
import jax.numpy as jnp
from jax import lax
from jax.experimental import pallas as pl
from jax.experimental.pallas import tpu as pltpu
from jax.experimental.pallas import tpu_sc as plsc

VOCAB_SIZE = 100000
HIDDEN = 64
PAD = 0
N_TOKENS = 4096 * 200
LANES = 16
NUM_CORES = 2
NUM_SUBCORES = 16
NUM_WORKERS = NUM_CORES * NUM_SUBCORES

IDS_PER_GATHER = 128
GATHERS_PER_CHUNK = 8
CHUNK = IDS_PER_GATHER * GATHERS_PER_CHUNK
ROWS128 = N_TOKENS // IDS_PER_GATHER
ROWS128_PER_WORKER = ROWS128 // NUM_WORKERS
CHUNKS_PER_WORKER = ROWS128_PER_WORKER // GATHERS_PER_CHUNK
GROUPS_PER_CHUNK = CHUNK // LANES


def _sc_body(ids_hbm, table_hbm, out_hbm, mask_hbm, idx_v, rows_v, mask_v, sem):
    wid = lax.axis_index("c") * NUM_SUBCORES + lax.axis_index("s")

    def chunk_body(k, carry):
        row0 = wid * ROWS128_PER_WORKER + k * GATHERS_PER_CHUNK
        pltpu.sync_copy(ids_hbm.at[pl.ds(row0, GATHERS_PER_CHUNK)], idx_v)

        descs = []
        for j in range(GATHERS_PER_CHUNK):
            descs.append(
                pltpu.async_copy(
                    table_hbm.at[idx_v.at[j]],
                    rows_v.at[pl.ds(j * IDS_PER_GATHER, IDS_PER_GATHER)],
                    sem,
                )
            )
        for d in descs:
            d.wait()

        for j in range(GATHERS_PER_CHUNK):
            vmin = None
            for c in range(IDS_PER_GATHER // LANES):
                idv = idx_v[j, pl.ds(c * LANES, LANES)]
                mask_v[j, pl.ds(c * LANES, LANES)] = jnp.minimum(idv, 1)
                vmin = idv if vmin is None else jnp.minimum(vmin, idv)
            smin = vmin[0]
            for r in range(1, LANES):
                smin = jnp.minimum(smin, vmin[r])

            @pl.when(smin == PAD)
            def _fixup(j=j):
                def fix_group(g, carry2):
                    idv = idx_v[j, pl.ds(g * LANES, LANES)]
                    base = j * IDS_PER_GATHER + g * LANES

                    for r in range(LANES):
                        @pl.when(idv[r] == PAD)
                        def _zero_row(r=r):
                            for c in range(HIDDEN // LANES):
                                rows_v[base + r, pl.ds(c * LANES, LANES)] = (
                                    jnp.zeros((LANES,), jnp.float32)
                                )
                    return carry2

                lax.fori_loop(0, IDS_PER_GATHER // LANES, fix_group, 0)

        pltpu.sync_copy(rows_v, out_hbm.at[pl.ds(row0 * IDS_PER_GATHER, CHUNK)])
        pltpu.sync_copy(mask_v, mask_hbm.at[pl.ds(row0, GATHERS_PER_CHUNK)])
        return carry

    lax.fori_loop(0, CHUNKS_PER_WORKER, chunk_body, 0)


@jax.jit
def kernel(input_ids, embed_weight):
    ids2d = input_ids.reshape(ROWS128, IDS_PER_GATHER)
    mesh = plsc.VectorSubcoreMesh(
        core_axis_name="c", subcore_axis_name="s",
        num_cores=NUM_CORES, num_subcores=NUM_SUBCORES,
    )
    out, mask2d = pl.kernel(
        _sc_body,
        out_type=[
            jax.ShapeDtypeStruct((N_TOKENS, HIDDEN), jnp.float32),
            jax.ShapeDtypeStruct((ROWS128, IDS_PER_GATHER), jnp.int32),
        ],
        mesh=mesh,
        compiler_params=pltpu.CompilerParams(use_tc_tiling_on_sc=False),
        scratch_types=[
            pltpu.VMEM((GATHERS_PER_CHUNK, IDS_PER_GATHER), jnp.int32),
            pltpu.VMEM((CHUNK, HIDDEN), jnp.float32),
            pltpu.VMEM((GATHERS_PER_CHUNK, IDS_PER_GATHER), jnp.int32),
            pltpu.SemaphoreType.DMA,
        ],
    )(ids2d, embed_weight)
    return (
        out.reshape(input_ids.shape[0], input_ids.shape[1], HIDDEN),
        mask2d.reshape(input_ids.shape),
    )


# speedup vs baseline: 4.2514x; 1.0728x over previous
import jax
import jax.numpy as jnp
from jax import lax
from jax.experimental import pallas as pl
from jax.experimental.pallas import tpu as pltpu
from jax.experimental.pallas import tpu_sc as plsc

VOCAB_SIZE = 100000
HIDDEN = 64
PAD = 0
N_TOKENS = 4096 * 200
LANES = 16
NUM_CORES = 2
NUM_SUBCORES = 16
NUM_WORKERS = NUM_CORES * NUM_SUBCORES

IDS_PER_GATHER = 128
GATHERS_PER_CHUNK = 4
CHUNK = IDS_PER_GATHER * GATHERS_PER_CHUNK
ROWS128 = N_TOKENS // IDS_PER_GATHER
ROWS128_PER_WORKER = ROWS128 // NUM_WORKERS
CHUNKS_PER_WORKER = ROWS128_PER_WORKER // GATHERS_PER_CHUNK


def _mask_and_fixup(idx_b, rows_b, mask_all, k):
    for j in range(GATHERS_PER_CHUNK):
        vmin = None
        for c in range(IDS_PER_GATHER // LANES):
            idv = idx_b[j, pl.ds(c * LANES, LANES)]
            mask_all[k * GATHERS_PER_CHUNK + j, pl.ds(c * LANES, LANES)] = (
                jnp.minimum(idv, 1)
            )
            vmin = idv if vmin is None else jnp.minimum(vmin, idv)
        smin = vmin[0]
        for r in range(1, LANES):
            smin = jnp.minimum(smin, vmin[r])

        @pl.when(smin == PAD)
        def _fixup(j=j):
            def fix_group(g, carry):
                idv = idx_b[j, pl.ds(g * LANES, LANES)]
                base = j * IDS_PER_GATHER + g * LANES

                for r in range(LANES):
                    @pl.when(idv[r] == PAD)
                    def _zero_row(r=r):
                        for c in range(HIDDEN // LANES):
                            rows_b[base + r, pl.ds(c * LANES, LANES)] = (
                                jnp.zeros((LANES,), jnp.float32)
                            )
                return carry

            lax.fori_loop(0, IDS_PER_GATHER // LANES, fix_group, 0)


def _sc_body(ids_hbm, table_hbm, out_hbm, mask_hbm,
             idx0, idx1, rows0, rows1, mask_all,
             gsem0, gsem1, osem0, osem1):
    wid = lax.axis_index("c") * NUM_SUBCORES + lax.axis_index("s")
    row128_0 = wid * ROWS128_PER_WORKER
    idx_bufs = (idx0, idx1)
    rows_bufs = (rows0, rows1)
    gsems = (gsem0, gsem1)
    osems = (osem0, osem1)

    def fire_gathers(k, b):
        descs = []
        for j in range(GATHERS_PER_CHUNK):
            descs.append(
                pltpu.async_copy(
                    table_hbm.at[idx_bufs[b].at[j]],
                    rows_bufs[b].at[pl.ds(j * IDS_PER_GATHER, IDS_PER_GATHER)],
                    gsems[b],
                )
            )
        return descs

    pltpu.sync_copy(ids_hbm.at[pl.ds(row128_0, GATHERS_PER_CHUNK)], idx0)
    fire_gathers(0, 0)

    def round_body(t, carry):
        for b in range(2):
            k = 2 * t + b
            bn = 1 - b

            @pl.when(k + 1 < CHUNKS_PER_WORKER)
            def _prefetch():
                pltpu.sync_copy(
                    ids_hbm.at[pl.ds(row128_0 + (k + 1) * GATHERS_PER_CHUNK,
                                     GATHERS_PER_CHUNK)],
                    idx_bufs[bn],
                )

                @pl.when(k >= 1)
                def _drain_out():
                    pltpu.make_async_copy(
                        rows_bufs[bn],
                        out_hbm.at[pl.ds(0, CHUNK)],
                        osems[bn],
                    ).wait()

                fire_gathers(k + 1, bn)

            for j in range(GATHERS_PER_CHUNK):
                pltpu.make_async_copy(
                    table_hbm.at[idx_bufs[b].at[j]],
                    rows_bufs[b].at[pl.ds(j * IDS_PER_GATHER, IDS_PER_GATHER)],
                    gsems[b],
                ).wait()

            _mask_and_fixup(idx_bufs[b], rows_bufs[b], mask_all, k)

            pltpu.async_copy(
                rows_bufs[b],
                out_hbm.at[pl.ds((row128_0 + k * GATHERS_PER_CHUNK)
                                 * IDS_PER_GATHER, CHUNK)],
                osems[b],
            )
        return carry

    lax.fori_loop(0, CHUNKS_PER_WORKER // 2, round_body, 0)

    for b in range(2):
        pltpu.make_async_copy(
            rows_bufs[b], out_hbm.at[pl.ds(0, CHUNK)], osems[b]
        ).wait()

    pltpu.sync_copy(mask_all, mask_hbm.at[pl.ds(row128_0, ROWS128_PER_WORKER)])


@jax.jit
def kernel(input_ids, embed_weight):
    ids2d = input_ids.reshape(ROWS128, IDS_PER_GATHER)
    mesh = plsc.VectorSubcoreMesh(
        core_axis_name="c", subcore_axis_name="s",
        num_cores=NUM_CORES, num_subcores=NUM_SUBCORES,
    )
    out, mask2d = pl.kernel(
        _sc_body,
        out_type=[
            jax.ShapeDtypeStruct((N_TOKENS, HIDDEN), jnp.float32),
            jax.ShapeDtypeStruct((ROWS128, IDS_PER_GATHER), jnp.int32),
        ],
        mesh=mesh,
        compiler_params=pltpu.CompilerParams(use_tc_tiling_on_sc=False),
        scratch_types=[
            pltpu.VMEM((GATHERS_PER_CHUNK, IDS_PER_GATHER), jnp.int32),
            pltpu.VMEM((GATHERS_PER_CHUNK, IDS_PER_GATHER), jnp.int32),
            pltpu.VMEM((CHUNK, HIDDEN), jnp.float32),
            pltpu.VMEM((CHUNK, HIDDEN), jnp.float32),
            pltpu.VMEM((ROWS128_PER_WORKER, IDS_PER_GATHER), jnp.int32),
            pltpu.SemaphoreType.DMA,
            pltpu.SemaphoreType.DMA,
            pltpu.SemaphoreType.DMA,
            pltpu.SemaphoreType.DMA,
        ],
    )(ids2d, embed_weight)
    return (
        out.reshape(input_ids.shape[0], input_ids.shape[1], HIDDEN),
        mask2d.reshape(input_ids.shape),
    )


# speedup vs baseline: 7.4982x; 1.7637x over previous
import jax
import jax.numpy as jnp
from jax import lax
from jax.experimental import pallas as pl
from jax.experimental.pallas import tpu as pltpu
from jax.experimental.pallas import tpu_sc as plsc

VOCAB_SIZE = 100000
HIDDEN = 64
PAD = 0
N_TOKENS = 4096 * 200
LANES = 16
NUM_CORES = 2
NUM_SUBCORES = 16
NUM_WORKERS = NUM_CORES * NUM_SUBCORES

IDS_PER_GATHER = 128
GATHERS_PER_CHUNK = 4
CHUNK = IDS_PER_GATHER * GATHERS_PER_CHUNK
ROWS128 = N_TOKENS // IDS_PER_GATHER
ROWS128_PER_WORKER = ROWS128 // NUM_WORKERS
CHUNKS_PER_WORKER = ROWS128_PER_WORKER // GATHERS_PER_CHUNK


def _mask_and_fixup(idx_b, rows_b, mask_all, k):
    for j in range(GATHERS_PER_CHUNK):
        vmin = None
        for c in range(IDS_PER_GATHER // LANES):
            idv = idx_b[j, pl.ds(c * LANES, LANES)]
            mask_all[k * GATHERS_PER_CHUNK + j, pl.ds(c * LANES, LANES)] = (
                jnp.minimum(idv, 1)
            )
            vmin = idv if vmin is None else jnp.minimum(vmin, idv)
        smin = vmin[0]
        for r in range(1, LANES):
            smin = jnp.minimum(smin, vmin[r])

        @pl.when(smin == PAD)
        def _fixup(j=j):
            def fix_group(g, carry):
                idv = idx_b[j, pl.ds(g * LANES, LANES)]
                base = j * IDS_PER_GATHER + g * LANES

                for r in range(LANES):
                    @pl.when(idv[r] == PAD)
                    def _zero_row(r=r):
                        for c in range(HIDDEN // LANES):
                            rows_b[base + r, pl.ds(c * LANES, LANES)] = (
                                jnp.zeros((LANES,), jnp.float32)
                            )
                return carry

            lax.fori_loop(0, IDS_PER_GATHER // LANES, fix_group, 0)


def _sc_body(ids_hbm, table_hbm, out_hbm, mask_hbm,
             idx0, idx1, rows0, rows1, mask_all,
             gsem0, gsem1, osem0, osem1):
    wid = lax.axis_index("c") * NUM_SUBCORES + lax.axis_index("s")
    row128_0 = wid * ROWS128_PER_WORKER
    idx_bufs = (idx0, idx1)
    rows_bufs = (rows0, rows1)
    gsems = (gsem0, gsem1)
    osems = (osem0, osem1)

    def fire_gathers(k, b):
        descs = []
        for j in range(GATHERS_PER_CHUNK):
            descs.append(
                pltpu.async_copy(
                    table_hbm.at[idx_bufs[b].at[j]],
                    rows_bufs[b].at[pl.ds(j * IDS_PER_GATHER, IDS_PER_GATHER)],
                    gsems[b],
                )
            )
        return descs

    pltpu.sync_copy(ids_hbm.at[pl.ds(row128_0, GATHERS_PER_CHUNK)], idx0)
    fire_gathers(0, 0)

    def round_body(t, carry):
        for b in range(2):
            k = 2 * t + b
            bn = 1 - b

            @pl.when(k + 1 < CHUNKS_PER_WORKER)
            def _prefetch():
                pltpu.sync_copy(
                    ids_hbm.at[pl.ds(row128_0 + (k + 1) * GATHERS_PER_CHUNK,
                                     GATHERS_PER_CHUNK)],
                    idx_bufs[bn],
                )

                @pl.when(k >= 1)
                def _drain_out():
                    pltpu.make_async_copy(
                        rows_bufs[bn],
                        out_hbm.at[pl.ds(0, CHUNK), pl.ds(0, HIDDEN)],
                        osems[bn],
                    ).wait()

                fire_gathers(k + 1, bn)

            for j in range(GATHERS_PER_CHUNK):
                pltpu.make_async_copy(
                    table_hbm.at[idx_bufs[b].at[j]],
                    rows_bufs[b].at[pl.ds(j * IDS_PER_GATHER, IDS_PER_GATHER)],
                    gsems[b],
                ).wait()

            _mask_and_fixup(idx_bufs[b], rows_bufs[b], mask_all, k)

            pltpu.async_copy(
                rows_bufs[b],
                out_hbm.at[pl.ds((row128_0 + k * GATHERS_PER_CHUNK)
                                 * IDS_PER_GATHER, CHUNK),
                           pl.ds(0, HIDDEN)],
                osems[b],
            )
        return carry

    lax.fori_loop(0, CHUNKS_PER_WORKER // 2, round_body, 0)

    for b in range(2):
        pltpu.make_async_copy(
            rows_bufs[b],
            out_hbm.at[pl.ds(0, CHUNK), pl.ds(0, HIDDEN)],
            osems[b],
        ).wait()

    pltpu.sync_copy(mask_all, mask_hbm.at[pl.ds(row128_0, ROWS128_PER_WORKER)])


@jax.jit
def kernel(input_ids, embed_weight):
    ids2d = input_ids.reshape(ROWS128, IDS_PER_GATHER)
    mesh = plsc.VectorSubcoreMesh(
        core_axis_name="c", subcore_axis_name="s",
        num_cores=NUM_CORES, num_subcores=NUM_SUBCORES,
    )
    out, mask2d = pl.kernel(
        _sc_body,
        out_type=[
            jax.ShapeDtypeStruct((N_TOKENS, 128), jnp.float32),
            jax.ShapeDtypeStruct((ROWS128, IDS_PER_GATHER), jnp.int32),
        ],
        mesh=mesh,
        compiler_params=pltpu.CompilerParams(use_tc_tiling_on_sc=False),
        scratch_types=[
            pltpu.VMEM((GATHERS_PER_CHUNK, IDS_PER_GATHER), jnp.int32),
            pltpu.VMEM((GATHERS_PER_CHUNK, IDS_PER_GATHER), jnp.int32),
            pltpu.VMEM((CHUNK, HIDDEN), jnp.float32),
            pltpu.VMEM((CHUNK, HIDDEN), jnp.float32),
            pltpu.VMEM((ROWS128_PER_WORKER, IDS_PER_GATHER), jnp.int32),
            pltpu.SemaphoreType.DMA,
            pltpu.SemaphoreType.DMA,
            pltpu.SemaphoreType.DMA,
            pltpu.SemaphoreType.DMA,
        ],
    )(ids2d, embed_weight)
    out3 = out.reshape(input_ids.shape[0], input_ids.shape[1], 128)
    return (
        lax.slice(out3, (0, 0, 0), (input_ids.shape[0], input_ids.shape[1],
                                    HIDDEN)),
        mask2d.reshape(input_ids.shape),
    )
